# baseline (speedup 1.0000x reference)
import functools

import jax
import jax.numpy as jnp
from jax import lax
from jax.experimental import pallas as pl
from jax.experimental.pallas import tpu as pltpu
from jax.experimental.pallas import tpu_sc as plsc


def kernel(_input, weights):
    bsz, seq_len = _input.shape
    embed_dim = weights.shape[1]

    info = plsc.get_sparse_core_info()
    nc, ns = info.num_cores, info.num_subcores
    nw = nc * ns
    rows_per_w = seq_len // nw
    chunk = 64
    n_chunks = rows_per_w // chunk

    mesh = plsc.VectorSubcoreMesh(core_axis_name="c", subcore_axis_name="s")

    @functools.partial(
        pl.kernel,
        mesh=mesh,
        out_type=jax.ShapeDtypeStruct((bsz * seq_len, embed_dim), jnp.float32),
        scratch_types=[
            pltpu.VMEM((chunk, embed_dim), jnp.float32),
            pltpu.SemaphoreType.DMA,
        ],
    )
    def k(w_hbm, out_hbm, buf, sem):
        wid = lax.axis_index("s") * nc + lax.axis_index("c")
        base = wid * rows_per_w
        for i in range(n_chunks):
            start = base + i * chunk
            pltpu.sync_copy(w_hbm.at[pl.ds(start, chunk)], buf)
            for b in range(bsz):
                pltpu.sync_copy(buf, out_hbm.at[pl.ds(b * seq_len + start, chunk)])

    out = k(weights)
    return out.reshape(bsz, seq_len, embed_dim)
